# baseline (speedup 1.0000x reference)
import functools

import jax
import jax.numpy as jnp
from jax import lax
from jax.experimental import pallas as pl
from jax.experimental.pallas import tpu as pltpu


def _fused_kernel(scal_ref, adj_ref, x_ref, w1_ref, w2_ref, out_ref,
                  xw_scr, adj_scr, t_scr, af_scr, *, num_steps, h1, h2,
                  inv_n, tma, tmb, ga, gtiles):
    i = pl.program_id(0)

    @pl.when(i == 0)
    def _():
        xw_scr[...] = jnp.dot(x_ref[...], w1_ref[...],
                              preferred_element_type=jnp.float32
                              ).astype(jnp.bfloat16)

    @pl.when(i < ga)
    def _():
        adj_scr[pl.ds(i * tma, tma), :] = adj_ref[...].astype(jnp.bfloat16)
        h = jnp.maximum(jnp.dot(adj_scr[pl.ds(i * tma, tma), :], xw_scr[...],
                                preferred_element_type=jnp.float32),
                        0.0).astype(jnp.bfloat16)
        for s in range(num_steps):
            ts = jnp.dot(h[:, s * h1:(s + 1) * h1], w2_ref[s],
                         preferred_element_type=jnp.float32)
            t_scr[pl.ds(i * tma, tma), s * h2:(s + 1) * h2] = (
                ts.astype(jnp.bfloat16))

    @pl.when(i == ga)
    def _():
        a = jnp.maximum(jnp.dot(adj_scr[...], t_scr[...],
                                preferred_element_type=jnp.float32), 0.0)
        a = a - jnp.sum(a, axis=0, keepdims=True) * inv_n
        af_scr[...] = a.astype(jnp.bfloat16)

    @pl.when(i >= ga)
    def _():
        tile = i - ga
        rows = af_scr[pl.ds(tile * tmb, tmb), :]
        ar = rows * jnp.bfloat16(0.5)
        logits = [lax.dot_general(ar[:, s * h2:(s + 1) * h2],
                                  af_scr[:, s * h2:(s + 1) * h2],
                                  (((1,), (1,)), ((), ())),
                                  preferred_element_type=jnp.float32)
                  for s in range(num_steps)]
        acc = None
        for s in range(num_steps):
            term = scal_ref[s] * jnp.tanh(logits[s])
            acc = term if acc is None else acc + term
        out_ref[...] = acc + scal_ref[num_steps]


def kernel(X, adj, w1_stack, w2_stack, sqrt_one_minus_alphas_cumprod,
           cumulative_sqrt_one_minus_alphas_cumprod):
    time_step, timesteps = 1, 4
    N, F_in = X.shape
    H1 = w1_stack.shape[-1]
    H2 = w2_stack.shape[-1]
    S = timesteps + 1 - time_step
    SH1, SH2 = S * H1, S * H2
    cdt = jnp.bfloat16

    denom = cumulative_sqrt_one_minus_alphas_cumprod[time_step - 1].astype(
        jnp.float32)
    coefs = (sqrt_one_minus_alphas_cumprod[time_step - 1: timesteps]
             .astype(jnp.float32) / denom)
    half_coefs = 0.5 * coefs
    scalars = jnp.concatenate([half_coefs, jnp.sum(half_coefs)[None]])

    Xb = X.astype(cdt)
    w1s = w1_stack[time_step: timesteps + 1].astype(cdt)
    w1_cat = jnp.transpose(w1s, (1, 0, 2)).reshape(F_in, SH1)
    w2s = w2_stack[time_step: timesteps + 1].astype(cdt)

    TMA = 640 if N % 640 == 0 else (512 if N % 512 == 0 else 256)
    TMB = 512 if N % 512 == 0 else 256
    GA = N // TMA
    GB = N // TMB

    out = pl.pallas_call(
        functools.partial(_fused_kernel, num_steps=S, h1=H1, h2=H2,
                          inv_n=1.0 / N, tma=TMA, tmb=TMB, ga=GA, gtiles=GB),
        out_shape=jax.ShapeDtypeStruct((N, N), jnp.float32),
        grid_spec=pltpu.PrefetchScalarGridSpec(
            num_scalar_prefetch=1,
            grid=(GA + GB,),
            in_specs=[
                pl.BlockSpec((TMA, N),
                             lambda i, scal: (jnp.minimum(i, GA - 1), 0)),
                pl.BlockSpec((N, F_in), lambda i, scal: (0, 0)),
                pl.BlockSpec((F_in, SH1), lambda i, scal: (0, 0)),
                pl.BlockSpec((S, H1, H2), lambda i, scal: (0, 0, 0)),
            ],
            out_specs=pl.BlockSpec(
                (TMB, N), lambda i, scal: (jnp.maximum(i - GA, 0), 0)),
            scratch_shapes=[pltpu.VMEM((N, SH1), cdt),
                            pltpu.VMEM((N, N), cdt),
                            pltpu.VMEM((N, SH2), cdt),
                            pltpu.VMEM((N, SH2), cdt)],
        ),
        compiler_params=pltpu.CompilerParams(
            dimension_semantics=("arbitrary",), vmem_limit_bytes=60000 * 1024),
    )(scalars, adj, Xb, w1_cat, w2s)

    return out


# speedup vs baseline: 1.0015x; 1.0015x over previous
import functools

import jax
import jax.numpy as jnp
from jax import lax
from jax.experimental import pallas as pl
from jax.experimental.pallas import tpu as pltpu


def _fused_kernel(scal_ref, adj_ref, x_ref, w1_ref, w2_ref, out_ref,
                  xw_scr, adj_scr, t_scr, af_scr, *, num_steps, h1, h2,
                  inv_n, tma, tmb, ga, gtiles):
    i = pl.program_id(0)

    @pl.when(i == 0)
    def _():
        xw_scr[...] = jnp.dot(x_ref[...], w1_ref[...],
                              preferred_element_type=jnp.float32
                              ).astype(jnp.bfloat16)

    @pl.when(i < ga)
    def _():
        adjb = adj_ref[...].astype(jnp.bfloat16)
        adj_scr[pl.ds(i * tma, tma), :] = adjb
        h = jnp.maximum(jnp.dot(adjb, xw_scr[...],
                                preferred_element_type=jnp.float32),
                        0.0).astype(jnp.bfloat16)
        for s in range(num_steps):
            ts = jnp.dot(h[:, s * h1:(s + 1) * h1], w2_ref[s],
                         preferred_element_type=jnp.float32)
            t_scr[pl.ds(i * tma, tma), s * h2:(s + 1) * h2] = (
                ts.astype(jnp.bfloat16))

    @pl.when(i == ga)
    def _():
        a = jnp.maximum(jnp.dot(adj_scr[...], t_scr[...],
                                preferred_element_type=jnp.float32), 0.0)
        a = a - jnp.sum(a, axis=0, keepdims=True) * inv_n
        af_scr[...] = a.astype(jnp.bfloat16)

    @pl.when(i >= ga)
    def _():
        tile = i - ga
        rows = af_scr[pl.ds(tile * tmb, tmb), :]
        ar = rows * jnp.bfloat16(0.5)
        logits = [lax.dot_general(ar[:, s * h2:(s + 1) * h2],
                                  af_scr[:, s * h2:(s + 1) * h2],
                                  (((1,), (1,)), ((), ())),
                                  preferred_element_type=jnp.float32)
                  for s in range(num_steps)]
        acc = None
        for s in range(num_steps):
            term = scal_ref[s] * jnp.tanh(logits[s])
            acc = term if acc is None else acc + term
        out_ref[...] = acc + scal_ref[num_steps]


def kernel(X, adj, w1_stack, w2_stack, sqrt_one_minus_alphas_cumprod,
           cumulative_sqrt_one_minus_alphas_cumprod):
    time_step, timesteps = 1, 4
    N, F_in = X.shape
    H1 = w1_stack.shape[-1]
    H2 = w2_stack.shape[-1]
    S = timesteps + 1 - time_step
    SH1, SH2 = S * H1, S * H2
    cdt = jnp.bfloat16

    denom = cumulative_sqrt_one_minus_alphas_cumprod[time_step - 1].astype(
        jnp.float32)
    coefs = (sqrt_one_minus_alphas_cumprod[time_step - 1: timesteps]
             .astype(jnp.float32) / denom)
    half_coefs = 0.5 * coefs
    scalars = jnp.concatenate([half_coefs, jnp.sum(half_coefs)[None]])

    Xb = X.astype(cdt)
    w1s = w1_stack[time_step: timesteps + 1].astype(cdt)
    w1_cat = jnp.transpose(w1s, (1, 0, 2)).reshape(F_in, SH1)
    w2s = w2_stack[time_step: timesteps + 1].astype(cdt)

    TMA = 640 if N % 640 == 0 else (512 if N % 512 == 0 else 256)
    TMB = 512 if N % 512 == 0 else 256
    GA = N // TMA
    GB = N // TMB

    out = pl.pallas_call(
        functools.partial(_fused_kernel, num_steps=S, h1=H1, h2=H2,
                          inv_n=1.0 / N, tma=TMA, tmb=TMB, ga=GA, gtiles=GB),
        out_shape=jax.ShapeDtypeStruct((N, N), jnp.float32),
        grid_spec=pltpu.PrefetchScalarGridSpec(
            num_scalar_prefetch=1,
            grid=(GA + GB,),
            in_specs=[
                pl.BlockSpec((TMA, N),
                             lambda i, scal: (jnp.minimum(i, GA - 1), 0)),
                pl.BlockSpec((N, F_in), lambda i, scal: (0, 0)),
                pl.BlockSpec((F_in, SH1), lambda i, scal: (0, 0)),
                pl.BlockSpec((S, H1, H2), lambda i, scal: (0, 0, 0)),
            ],
            out_specs=pl.BlockSpec(
                (TMB, N), lambda i, scal: (jnp.maximum(i - GA, 0), 0)),
            scratch_shapes=[pltpu.VMEM((N, SH1), cdt),
                            pltpu.VMEM((N, N), cdt),
                            pltpu.VMEM((N, SH2), cdt),
                            pltpu.VMEM((N, SH2), cdt)],
        ),
        compiler_params=pltpu.CompilerParams(
            dimension_semantics=("arbitrary",), vmem_limit_bytes=60000 * 1024),
    )(scalars, adj, Xb, w1_cat, w2s)

    return out
